# baseline (speedup 1.0000x reference)
import functools

import jax
import jax.numpy as jnp
from jax import lax
from jax.experimental import pallas as pl
from jax.experimental.pallas import tpu as pltpu
from jax.experimental.pallas import tpu_sc as plsc

L_ROWS = 4096
M = 200
M1 = M - 1
EMB_DIM = 32

NC = 2
NS = 16
NW = NC * NS
CHUNK = 128
NCB = L_ROWS // CHUNK
PAIRS = M1 * NCB
PPW = PAIRS // NW
NTILE = EMB_DIM // 8

TB = 256


def _tc_body(obs_ref, src_ref, hiidx_ref, xi_ref, xjx_ref, xjy_ref):
    info = obs_ref[:, 0, :]
    rx = obs_ref[:, 3, :]
    ry = obs_ref[:, 4, :]

    iota = lax.broadcasted_iota(jnp.int32, (TB, M), 1)
    masked = jnp.where(info == 1.0, iota, M)
    first_one = jnp.min(masked, axis=1, keepdims=True)
    self_idx = jnp.where(first_one == M, 0, first_one)

    at_self = iota == self_idx
    info_i = info.astype(jnp.int32)
    hiidx_ref[...] = jnp.sum(jnp.where(at_self, info_i, 0), axis=1, keepdims=True)
    xi_x = jnp.sum(jnp.where(at_self, rx, 0.0), axis=1, keepdims=True)
    xi_y = jnp.sum(jnp.where(at_self, ry, 0.0), axis=1, keepdims=True)
    xi_ref[...] = jnp.concatenate([xi_x, xi_y], axis=1)

    keep_left = lax.broadcasted_iota(jnp.int32, (TB, M1), 1) < self_idx
    src = jnp.where(keep_left, info_i[:, :M1], info_i[:, 1:])
    src_ref[...] = src.T
    xjx_ref[...] = jnp.where(keep_left, rx[:, :M1], rx[:, 1:])
    xjy_ref[...] = jnp.where(keep_left, ry[:, :M1], ry[:, 1:])


def _tc_prepare(obs3):
    grid = (L_ROWS // TB,)
    return pl.pallas_call(
        _tc_body,
        grid=grid,
        in_specs=[pl.BlockSpec((TB, 5, M), lambda i: (i, 0, 0))],
        out_specs=[
            pl.BlockSpec((M1, TB), lambda i: (0, i)),
            pl.BlockSpec((TB, 1), lambda i: (i, 0)),
            pl.BlockSpec((TB, 2), lambda i: (i, 0)),
            pl.BlockSpec((TB, M1), lambda i: (i, 0)),
            pl.BlockSpec((TB, M1), lambda i: (i, 0)),
        ],
        out_shape=[
            jax.ShapeDtypeStruct((M1, L_ROWS), jnp.int32),
            jax.ShapeDtypeStruct((L_ROWS, 1), jnp.int32),
            jax.ShapeDtypeStruct((L_ROWS, 2), jnp.float32),
            jax.ShapeDtypeStruct((L_ROWS, M1), jnp.float32),
            jax.ShapeDtypeStruct((L_ROWS, M1), jnp.float32),
        ],
    )(obs3)


def _transpose_chunk(gbuf, tbuf, eidx, zeros):
    def blk(i16, _):
        for bs in range(16):
            b = i16 * 16 + bs
            bvec = zeros + b
            for h in range(2):
                v = gbuf[b, pl.ds(h * 16, 16)]
                plsc.store_scatter(tbuf, [eidx + h * 16, bvec], v)
        return 0

    lax.fori_loop(0, CHUNK // 16, blk, 0)


def _sc_body(table_hbm, srct_hbm, hiidx_hbm, hj_hbm, hi_hbm,
             idx_v, cidx_v, gbuf, tbuf, hgbuf, gsem, ssem):
    wid = lax.axis_index("s") * NC + lax.axis_index("c")
    eidx = lax.iota(jnp.int32, 16)
    zeros = jnp.zeros((16,), jnp.int32)

    pltpu.sync_copy(srct_hbm.at[wid], idx_v)

    qbase = wid * PPW

    def gdesc(t, gslot):
        return pltpu.make_async_copy(
            table_hbm.at[idx_v.at[t]], gbuf.at[gslot], gsem)

    def sdesc(t, tslot, r):
        q = qbase + t
        return pltpu.make_async_copy(
            tbuf.at[tslot, pl.ds(r * 8, 8), pl.ds(0, 128)],
            hj_hbm.at[q >> 5, r, q & 31], ssem)

    for u in range(4):
        gdesc(u, u).start()

    def step(k, _):
        for u in range(4):
            t = 4 * k + u
            live = t <= PPW - 1

            @pl.when(live)
            def _():
                gdesc(t, u).wait()

            @pl.when(jnp.logical_and(t >= 2, live))
            def _():
                for r in range(NTILE):
                    sdesc(t - 2, u % 2, r).wait()

            @pl.when(live)
            def _():
                _transpose_chunk(gbuf.at[u], tbuf.at[u % 2], eidx, zeros)
                for r in range(NTILE):
                    sdesc(t, u % 2, r).start()

            @pl.when(t + 4 <= PPW - 1)
            def _():
                gdesc(t + 4, u).start()

        return 0

    lax.fori_loop(0, (PPW + 3) // 4, step, 0)

    for t in (PPW - 2, PPW - 1):
        for r in range(NTILE):
            sdesc(t, t % 2, r).wait()

    pltpu.sync_copy(hiidx_hbm.at[wid], cidx_v)
    pltpu.async_copy(table_hbm.at[cidx_v], hgbuf, gsem).wait()
    _transpose_chunk(hgbuf, tbuf.at[0], eidx, zeros)
    for r in range(NTILE):
        pltpu.make_async_copy(tbuf.at[0, pl.ds(r * 8, 8), pl.ds(0, 128)],
                              hi_hbm.at[r, wid], ssem).start()
    for r in range(NTILE):
        pltpu.make_async_copy(tbuf.at[0, pl.ds(r * 8, 8), pl.ds(0, 128)],
                              hi_hbm.at[r, wid], ssem).wait()


@functools.cache
def _sc_gather():
    return pl.kernel(
        _sc_body,
        out_type=[
            jax.ShapeDtypeStruct((M1, NTILE, NCB, 8, 128), jnp.float32),
            jax.ShapeDtypeStruct((NTILE, NCB, 8, 128), jnp.float32),
        ],
        mesh=plsc.VectorSubcoreMesh(core_axis_name="c", subcore_axis_name="s",
                                    num_cores=NC, num_subcores=NS),
        compiler_params=pltpu.CompilerParams(use_tc_tiling_on_sc=False,
                                             needs_layout_passes=False),
        scratch_types=[
            pltpu.VMEM((PPW, CHUNK), jnp.int32),
            pltpu.VMEM((CHUNK,), jnp.int32),
            pltpu.VMEM((4, CHUNK, EMB_DIM), jnp.float32),
            pltpu.VMEM((2, EMB_DIM, 129), jnp.float32),
            pltpu.VMEM((CHUNK, EMB_DIM), jnp.float32),
            pltpu.SemaphoreType.DMA,
            pltpu.SemaphoreType.DMA,
        ],
    )


def kernel(obs, emb_weight):
    obs3 = obs.reshape(L_ROWS, 5, M)
    srct, hiidx, xi, xjx, xjy = _tc_prepare(obs3)
    srct3 = srct.reshape(NW, PPW, CHUNK)
    hiidx2 = hiidx.reshape(NW, CHUNK)
    hj5, hi4 = _sc_gather()(emb_weight, srct3, hiidx2)
    hj = hj5.transpose(2, 4, 0, 1, 3).reshape(L_ROWS, M1, EMB_DIM)
    hi = hi4.transpose(1, 3, 0, 2).reshape(L_ROWS, EMB_DIM)
    xj = jnp.stack([xjx, xjy], axis=-1)
    return (hi, xi, hj, xj)


# speedup vs baseline: 1.1100x; 1.1100x over previous
import functools

import jax
import jax.numpy as jnp
from jax import lax
from jax.experimental import pallas as pl
from jax.experimental.pallas import tpu as pltpu
from jax.experimental.pallas import tpu_sc as plsc

L_ROWS = 4096
M = 200
M1 = M - 1
EMB_DIM = 32

NC = 2
NS = 16
NW = NC * NS
CHUNK = 128
NCB = L_ROWS // CHUNK
PAIRS = M1 * NCB
PPW = PAIRS // NW
NTILE = EMB_DIM // 8

TB = 256


def _tc_body(obs_ref, src_ref, hiidx_ref, xi_ref, xjx_ref, xjy_ref):
    info = obs_ref[:, 0, :]
    rx = obs_ref[:, 3, :]
    ry = obs_ref[:, 4, :]

    iota = lax.broadcasted_iota(jnp.int32, (TB, M), 1)
    masked = jnp.where(info == 1.0, iota, M)
    first_one = jnp.min(masked, axis=1, keepdims=True)
    self_idx = jnp.where(first_one == M, 0, first_one)

    at_self = iota == self_idx
    info_i = info.astype(jnp.int32)
    hiidx_ref[...] = jnp.sum(jnp.where(at_self, info_i, 0), axis=1, keepdims=True)
    xi_x = jnp.sum(jnp.where(at_self, rx, 0.0), axis=1, keepdims=True)
    xi_y = jnp.sum(jnp.where(at_self, ry, 0.0), axis=1, keepdims=True)
    xi_ref[...] = jnp.concatenate([xi_x, xi_y], axis=1)

    keep_left = lax.broadcasted_iota(jnp.int32, (TB, M1), 1) < self_idx
    src = jnp.where(keep_left, info_i[:, :M1], info_i[:, 1:])
    src_ref[...] = jnp.concatenate([src, src[:, :1]], axis=1).T
    xjx_ref[...] = jnp.where(keep_left, rx[:, :M1], rx[:, 1:])
    xjy_ref[...] = jnp.where(keep_left, ry[:, :M1], ry[:, 1:])


def _tc_prepare(obs3):
    grid = (L_ROWS // TB,)
    return pl.pallas_call(
        _tc_body,
        grid=grid,
        in_specs=[pl.BlockSpec((TB, 5, M), lambda i: (i, 0, 0))],
        out_specs=[
            pl.BlockSpec((M, TB), lambda i: (0, i)),
            pl.BlockSpec((TB, 1), lambda i: (i, 0)),
            pl.BlockSpec((TB, 2), lambda i: (i, 0)),
            pl.BlockSpec((TB, M1), lambda i: (i, 0)),
            pl.BlockSpec((TB, M1), lambda i: (i, 0)),
        ],
        out_shape=[
            jax.ShapeDtypeStruct((M, L_ROWS), jnp.int32),
            jax.ShapeDtypeStruct((L_ROWS, 1), jnp.int32),
            jax.ShapeDtypeStruct((L_ROWS, 2), jnp.float32),
            jax.ShapeDtypeStruct((L_ROWS, M1), jnp.float32),
            jax.ShapeDtypeStruct((L_ROWS, M1), jnp.float32),
        ],
    )(obs3)


def _transpose_chunk(gbuf, tbuf, eidx, zeros):
    def blk(i8, _):
        for bs in range(8):
            b = i8 * 8 + bs
            bvec = zeros + b
            for h in range(2):
                v = gbuf[b, pl.ds(h * 16, 16)]
                plsc.store_scatter(tbuf, [eidx + h * 16, bvec], v)
        return 0

    lax.fori_loop(0, CHUNK // 8, blk, 0)


def _sc_body(table_hbm, srct_hbm, hiidx_hbm, hj_hbm, hi_hbm,
             idx_v, cidx_v, gbuf, tbuf, hgbuf, gsem, ssem):
    wid = lax.axis_index("s") * NC + lax.axis_index("c")
    eidx = lax.iota(jnp.int32, 16)
    zeros = jnp.zeros((16,), jnp.int32)

    pltpu.sync_copy(srct_hbm.at[pl.ds(wid * PPW, PPW)], idx_v)

    qbase = wid * PPW

    def gdesc(t, gslot):
        return pltpu.make_async_copy(
            table_hbm.at[idx_v.at[t]], gbuf.at[gslot], gsem)

    def sdesc(t, tslot, r):
        q = qbase + t
        return pltpu.make_async_copy(
            tbuf.at[tslot, pl.ds(r * 8, 8), pl.ds(0, 128)],
            hj_hbm.at[q >> 5, r, q & 31], ssem)

    for u in range(4):
        gdesc(u, u).start()

    def step(k, _):
        for u in range(4):
            t = 4 * k + u
            live = t <= PPW - 1

            @pl.when(live)
            def _():
                gdesc(t, u).wait()

            @pl.when(jnp.logical_and(t >= 2, live))
            def _():
                for r in range(NTILE):
                    sdesc(t - 2, u % 2, r).wait()

            @pl.when(live)
            def _():
                _transpose_chunk(gbuf.at[u], tbuf.at[u % 2], eidx, zeros)
                for r in range(NTILE):
                    sdesc(t, u % 2, r).start()

            @pl.when(t + 4 <= PPW - 1)
            def _():
                gdesc(t + 4, u).start()

        return 0

    lax.fori_loop(0, (PPW + 3) // 4, step, 0)

    for t in (PPW - 2, PPW - 1):
        for r in range(NTILE):
            sdesc(t, t % 2, r).wait()

    pltpu.sync_copy(hiidx_hbm.at[wid], cidx_v)
    pltpu.async_copy(table_hbm.at[cidx_v], hgbuf, gsem).wait()
    _transpose_chunk(hgbuf, tbuf.at[0], eidx, zeros)
    for r in range(NTILE):
        pltpu.make_async_copy(tbuf.at[0, pl.ds(r * 8, 8), pl.ds(0, 128)],
                              hi_hbm.at[r, wid], ssem).start()
    for r in range(NTILE):
        pltpu.make_async_copy(tbuf.at[0, pl.ds(r * 8, 8), pl.ds(0, 128)],
                              hi_hbm.at[r, wid], ssem).wait()


@functools.cache
def _sc_gather():
    return pl.kernel(
        _sc_body,
        out_type=[
            jax.ShapeDtypeStruct((M1, NTILE, NCB, 8, 128), jnp.float32),
            jax.ShapeDtypeStruct((NTILE, NCB, 8, 128), jnp.float32),
        ],
        mesh=plsc.VectorSubcoreMesh(core_axis_name="c", subcore_axis_name="s",
                                    num_cores=NC, num_subcores=NS),
        compiler_params=pltpu.CompilerParams(use_tc_tiling_on_sc=False,
                                             needs_layout_passes=False),
        scratch_types=[
            pltpu.VMEM((PPW, CHUNK), jnp.int32),
            pltpu.VMEM((CHUNK,), jnp.int32),
            pltpu.VMEM((4, CHUNK, EMB_DIM), jnp.float32),
            pltpu.VMEM((2, EMB_DIM, 129), jnp.float32),
            pltpu.VMEM((CHUNK, EMB_DIM), jnp.float32),
            pltpu.SemaphoreType.DMA,
            pltpu.SemaphoreType.DMA,
        ],
    )


def kernel(obs, emb_weight):
    obs3 = obs.reshape(L_ROWS, 5, M)
    srct, hiidx, xi, xjx, xjy = _tc_prepare(obs3)
    srct3 = srct.reshape(M * NCB, CHUNK)
    hiidx2 = hiidx.reshape(NW, CHUNK)
    hj5, hi4 = _sc_gather()(emb_weight, srct3, hiidx2)
    hj = hj5.transpose(2, 4, 0, 1, 3).reshape(L_ROWS, M1, EMB_DIM)
    hi = hi4.transpose(1, 3, 0, 2).reshape(L_ROWS, EMB_DIM)
    xj = jnp.stack([xjx, xjy], axis=-1)
    return (hi, xi, hj, xj)


# speedup vs baseline: 1.1177x; 1.0069x over previous
import functools

import jax
import jax.numpy as jnp
from jax import lax
from jax.experimental import pallas as pl
from jax.experimental.pallas import tpu as pltpu
from jax.experimental.pallas import tpu_sc as plsc

L_ROWS = 4096
M = 200
M1 = M - 1
EMB_DIM = 32

NC = 2
NS = 16
NW = NC * NS
CHUNK = 128
NCB = L_ROWS // CHUNK
PAIRS = M1 * NCB
PPW = PAIRS // NW
NTILE = EMB_DIM // 8

TB = 256


def _tc_body(obs_ref, src_ref, hiidx_ref, xi_ref, xjx_ref, xjy_ref):
    info = obs_ref[:, 0, :]
    rx = obs_ref[:, 3, :]
    ry = obs_ref[:, 4, :]

    iota = lax.broadcasted_iota(jnp.int32, (TB, M), 1)
    masked = jnp.where(info == 1.0, iota, M)
    first_one = jnp.min(masked, axis=1, keepdims=True)
    self_idx = jnp.where(first_one == M, 0, first_one)

    at_self = iota == self_idx
    info_i = info.astype(jnp.int32)
    hiidx_ref[...] = jnp.sum(jnp.where(at_self, info_i, 0), axis=1, keepdims=True)
    xi_x = jnp.sum(jnp.where(at_self, rx, 0.0), axis=1, keepdims=True)
    xi_y = jnp.sum(jnp.where(at_self, ry, 0.0), axis=1, keepdims=True)
    xi_ref[...] = jnp.concatenate([xi_x, xi_y], axis=1)

    keep_left = lax.broadcasted_iota(jnp.int32, (TB, M1), 1) < self_idx
    src = jnp.where(keep_left, info_i[:, :M1], info_i[:, 1:])
    src_ref[...] = jnp.concatenate([src, src[:, :1]], axis=1).T
    xjx_ref[...] = jnp.where(keep_left, rx[:, :M1], rx[:, 1:])
    xjy_ref[...] = jnp.where(keep_left, ry[:, :M1], ry[:, 1:])


def _tc_prepare(obs3):
    grid = (L_ROWS // TB,)
    return pl.pallas_call(
        _tc_body,
        grid=grid,
        in_specs=[pl.BlockSpec((TB, 5, M), lambda i: (i, 0, 0))],
        out_specs=[
            pl.BlockSpec((M, TB), lambda i: (0, i)),
            pl.BlockSpec((TB, 1), lambda i: (i, 0)),
            pl.BlockSpec((TB, 2), lambda i: (i, 0)),
            pl.BlockSpec((TB, M1), lambda i: (i, 0)),
            pl.BlockSpec((TB, M1), lambda i: (i, 0)),
        ],
        out_shape=[
            jax.ShapeDtypeStruct((M, L_ROWS), jnp.int32),
            jax.ShapeDtypeStruct((L_ROWS, 1), jnp.int32),
            jax.ShapeDtypeStruct((L_ROWS, 2), jnp.float32),
            jax.ShapeDtypeStruct((L_ROWS, M1), jnp.float32),
            jax.ShapeDtypeStruct((L_ROWS, M1), jnp.float32),
        ],
    )(obs3)


def _transpose_chunk(gbuf, tbuf, eidx, zeros):
    def blk(i8, _):
        for bs in range(8):
            b = i8 * 8 + bs
            bvec = zeros + b
            for h in range(2):
                v = gbuf[b, pl.ds(h * 16, 16)]
                e = eidx + h * 16
                plsc.store_scatter(tbuf, [e >> 3, e & 7, bvec], v)
        return 0

    lax.fori_loop(0, CHUNK // 8, blk, 0)


def _sc_body(table_hbm, srct_hbm, hiidx_hbm, hj_hbm, hi_hbm,
             idx_v, cidx_v, gbuf, tbuf, hgbuf, gsem, ssem):
    wid = lax.axis_index("s") * NC + lax.axis_index("c")
    eidx = lax.iota(jnp.int32, 16)
    zeros = jnp.zeros((16,), jnp.int32)

    pltpu.sync_copy(srct_hbm.at[pl.ds(wid * PPW, PPW)], idx_v)

    qbase = wid * PPW

    def gdesc(t, gslot):
        return pltpu.make_async_copy(
            table_hbm.at[idx_v.at[t]], gbuf.at[gslot], gsem)

    def sdesc(t, tslot):
        q = qbase + t
        return pltpu.make_async_copy(
            tbuf.at[tslot, :, :, pl.ds(0, 128)],
            hj_hbm.at[q >> 5, :, q & 31], ssem)

    for u in range(4):
        gdesc(u, u).start()

    def step(k, _):
        for u in range(4):
            t = 4 * k + u
            live = t <= PPW - 1

            @pl.when(live)
            def _():
                gdesc(t, u).wait()

            @pl.when(jnp.logical_and(t >= 2, live))
            def _():
                sdesc(t - 2, u % 2).wait()

            @pl.when(live)
            def _():
                _transpose_chunk(gbuf.at[u], tbuf.at[u % 2], eidx, zeros)
                sdesc(t, u % 2).start()

            @pl.when(t + 4 <= PPW - 1)
            def _():
                gdesc(t + 4, u).start()

        return 0

    lax.fori_loop(0, (PPW + 3) // 4, step, 0)

    for t in (PPW - 2, PPW - 1):
        sdesc(t, t % 2).wait()

    pltpu.sync_copy(hiidx_hbm.at[wid], cidx_v)
    pltpu.async_copy(table_hbm.at[cidx_v], hgbuf, gsem).wait()
    _transpose_chunk(hgbuf, tbuf.at[0], eidx, zeros)
    pltpu.make_async_copy(tbuf.at[0, :, :, pl.ds(0, 128)],
                          hi_hbm.at[:, wid], ssem).start()
    pltpu.make_async_copy(tbuf.at[0, :, :, pl.ds(0, 128)],
                          hi_hbm.at[:, wid], ssem).wait()


@functools.cache
def _sc_gather():
    return pl.kernel(
        _sc_body,
        out_type=[
            jax.ShapeDtypeStruct((M1, NTILE, NCB, 8, 128), jnp.float32),
            jax.ShapeDtypeStruct((NTILE, NCB, 8, 128), jnp.float32),
        ],
        mesh=plsc.VectorSubcoreMesh(core_axis_name="c", subcore_axis_name="s",
                                    num_cores=NC, num_subcores=NS),
        compiler_params=pltpu.CompilerParams(use_tc_tiling_on_sc=False,
                                             needs_layout_passes=False),
        scratch_types=[
            pltpu.VMEM((PPW, CHUNK), jnp.int32),
            pltpu.VMEM((CHUNK,), jnp.int32),
            pltpu.VMEM((4, CHUNK, EMB_DIM), jnp.float32),
            pltpu.VMEM((2, NTILE, 8, 129), jnp.float32),
            pltpu.VMEM((CHUNK, EMB_DIM), jnp.float32),
            pltpu.SemaphoreType.DMA,
            pltpu.SemaphoreType.DMA,
        ],
    )


def kernel(obs, emb_weight):
    obs3 = obs.reshape(L_ROWS, 5, M)
    srct, hiidx, xi, xjx, xjy = _tc_prepare(obs3)
    srct3 = srct.reshape(M * NCB, CHUNK)
    hiidx2 = hiidx.reshape(NW, CHUNK)
    hj5, hi4 = _sc_gather()(emb_weight, srct3, hiidx2)
    hj = hj5.transpose(2, 4, 0, 1, 3).reshape(L_ROWS, M1, EMB_DIM)
    hi = hi4.transpose(1, 3, 0, 2).reshape(L_ROWS, EMB_DIM)
    xj = jnp.stack([xjx, xjy], axis=-1)
    return (hi, xi, hj, xj)


# speedup vs baseline: 1.2316x; 1.1019x over previous
import functools

import jax
import jax.numpy as jnp
from jax import lax
from jax.experimental import pallas as pl
from jax.experimental.pallas import tpu as pltpu
from jax.experimental.pallas import tpu_sc as plsc

L_ROWS = 4096
M = 200
M1 = M - 1
EMB_DIM = 32

NC = 2
NS = 16
NW = NC * NS
CHUNK = 128
NCB = L_ROWS // CHUNK
PAIRS = M1 * NCB
PPW = PAIRS // NW
NTILE = EMB_DIM // 8

TB = 512


def _tc_body(obs_ref, src_ref, hiidx_ref, xi_ref, xjx_ref, xjy_ref):
    info = obs_ref[0]
    rx = obs_ref[3]
    ry = obs_ref[4]

    iota = lax.broadcasted_iota(jnp.int32, (M, TB), 0)
    masked = jnp.where(info == 1.0, iota, M)
    first_one = jnp.min(masked, axis=0, keepdims=True)
    self_idx = jnp.where(first_one == M, 0, first_one)

    at_self = iota == self_idx
    info_i = info.astype(jnp.int32)
    hiidx_ref[...] = jnp.sum(jnp.where(at_self, info_i, 0), axis=0, keepdims=True)
    xi_x = jnp.sum(jnp.where(at_self, rx, 0.0), axis=0, keepdims=True)
    xi_y = jnp.sum(jnp.where(at_self, ry, 0.0), axis=0, keepdims=True)
    xi_ref[...] = jnp.concatenate([xi_x, xi_y], axis=0)

    keep_left = lax.broadcasted_iota(jnp.int32, (M1, TB), 0) < self_idx
    src = jnp.where(keep_left, info_i[:M1], info_i[1:])
    src_ref[...] = jnp.concatenate([src, src[:1]], axis=0)
    xjx_ref[...] = jnp.where(keep_left, rx[:M1], rx[1:])
    xjy_ref[...] = jnp.where(keep_left, ry[:M1], ry[1:])


def _tc_prepare(obs3t):
    grid = (L_ROWS // TB,)
    return pl.pallas_call(
        _tc_body,
        grid=grid,
        in_specs=[pl.BlockSpec((5, M, TB), lambda i: (0, 0, i))],
        out_specs=[
            pl.BlockSpec((M, TB), lambda i: (0, i)),
            pl.BlockSpec((1, TB), lambda i: (0, i)),
            pl.BlockSpec((2, TB), lambda i: (0, i)),
            pl.BlockSpec((M1, TB), lambda i: (0, i)),
            pl.BlockSpec((M1, TB), lambda i: (0, i)),
        ],
        out_shape=[
            jax.ShapeDtypeStruct((M, L_ROWS), jnp.int32),
            jax.ShapeDtypeStruct((1, L_ROWS), jnp.int32),
            jax.ShapeDtypeStruct((2, L_ROWS), jnp.float32),
            jax.ShapeDtypeStruct((M1, L_ROWS), jnp.float32),
            jax.ShapeDtypeStruct((M1, L_ROWS), jnp.float32),
        ],
    )(obs3t)


def _transpose_chunk(gbuf, tbuf, eidx, zeros):
    def blk(i8, _):
        for bs in range(8):
            b = i8 * 8 + bs
            bvec = zeros + b
            for h in range(2):
                v = gbuf[b, pl.ds(h * 16, 16)]
                e = eidx + h * 16
                plsc.store_scatter(tbuf, [e >> 3, e & 7, bvec], v)
        return 0

    lax.fori_loop(0, CHUNK // 8, blk, 0)


def _sc_body(table_hbm, srct_hbm, hiidx_hbm, hj_hbm, hi_hbm,
             idx_v, cidx_v, gbuf, tbuf, hgbuf, gsem, ssem):
    wid = lax.axis_index("s") * NC + lax.axis_index("c")
    eidx = lax.iota(jnp.int32, 16)
    zeros = jnp.zeros((16,), jnp.int32)

    pltpu.sync_copy(srct_hbm.at[pl.ds(wid * PPW, PPW)], idx_v)

    qbase = wid * PPW

    def gdesc(t, gslot):
        return pltpu.make_async_copy(
            table_hbm.at[idx_v.at[t]], gbuf.at[gslot], gsem)

    def sdesc(t, tslot):
        q = qbase + t
        return pltpu.make_async_copy(
            tbuf.at[tslot, :, :, pl.ds(0, 128)],
            hj_hbm.at[q >> 5, :, q & 31], ssem)

    for u in range(6):
        gdesc(u, u).start()

    def step(k, _):
        for u in range(6):
            t = 6 * k + u
            live = t <= PPW - 1

            @pl.when(live)
            def _():
                gdesc(t, u).wait()

            @pl.when(jnp.logical_and(t >= 2, live))
            def _():
                sdesc(t - 2, u % 2).wait()

            @pl.when(live)
            def _():
                _transpose_chunk(gbuf.at[u], tbuf.at[u % 2], eidx, zeros)
                sdesc(t, u % 2).start()

            @pl.when(t + 6 <= PPW - 1)
            def _():
                gdesc(t + 6, u).start()

        return 0

    lax.fori_loop(0, (PPW + 5) // 6, step, 0)

    for t in (PPW - 2, PPW - 1):
        sdesc(t, t % 2).wait()

    pltpu.sync_copy(hiidx_hbm.at[wid], cidx_v)
    pltpu.async_copy(table_hbm.at[cidx_v], hgbuf, gsem).wait()
    _transpose_chunk(hgbuf, tbuf.at[0], eidx, zeros)
    pltpu.make_async_copy(tbuf.at[0, :, :, pl.ds(0, 128)],
                          hi_hbm.at[:, wid], ssem).start()
    pltpu.make_async_copy(tbuf.at[0, :, :, pl.ds(0, 128)],
                          hi_hbm.at[:, wid], ssem).wait()


@functools.cache
def _sc_gather():
    return pl.kernel(
        _sc_body,
        out_type=[
            jax.ShapeDtypeStruct((M1, NTILE, NCB, 8, 128), jnp.float32),
            jax.ShapeDtypeStruct((NTILE, NCB, 8, 128), jnp.float32),
        ],
        mesh=plsc.VectorSubcoreMesh(core_axis_name="c", subcore_axis_name="s",
                                    num_cores=NC, num_subcores=NS),
        compiler_params=pltpu.CompilerParams(use_tc_tiling_on_sc=False,
                                             needs_layout_passes=False),
        scratch_types=[
            pltpu.VMEM((PPW, CHUNK), jnp.int32),
            pltpu.VMEM((CHUNK,), jnp.int32),
            pltpu.VMEM((6, CHUNK, EMB_DIM), jnp.float32),
            pltpu.VMEM((2, NTILE, 8, 129), jnp.float32),
            pltpu.VMEM((CHUNK, EMB_DIM), jnp.float32),
            pltpu.SemaphoreType.DMA,
            pltpu.SemaphoreType.DMA,
        ],
    )


def kernel(obs, emb_weight):
    obs3t = jnp.transpose(obs.reshape(L_ROWS, 5, M), (1, 2, 0))
    srct, hiidx, xi_t, xjxt, xjyt = _tc_prepare(obs3t)
    srct3 = srct.reshape(M * NCB, CHUNK)
    hiidx2 = hiidx.reshape(NW, CHUNK)
    hj5, hi4 = _sc_gather()(emb_weight, srct3, hiidx2)
    hj = hj5.transpose(2, 4, 0, 1, 3).reshape(L_ROWS, M1, EMB_DIM)
    hi = hi4.transpose(1, 3, 0, 2).reshape(L_ROWS, EMB_DIM)
    xi = xi_t.T
    xj = jnp.stack([xjxt, xjyt], axis=1).transpose(2, 0, 1)
    return (hi, xi, hj, xj)


# speedup vs baseline: 1.2613x; 1.0241x over previous
import functools

import jax
import jax.numpy as jnp
from jax import lax
from jax.experimental import pallas as pl
from jax.experimental.pallas import tpu as pltpu
from jax.experimental.pallas import tpu_sc as plsc

L_ROWS = 4096
M = 200
M1 = M - 1
EMB_DIM = 32

NC = 2
NS = 16
NW = NC * NS
CHUNK = 128
NCB = L_ROWS // CHUNK
PAIRS = M1 * NCB
PPW = PAIRS // NW
NTILE = EMB_DIM // 8

TB = 512


def _tc_body(obs_ref, src_ref, hiidx_ref, xi_ref, xjx_ref, xjy_ref):
    info = obs_ref[0]
    rx = obs_ref[3]
    ry = obs_ref[4]

    iota = lax.broadcasted_iota(jnp.int32, (M, TB), 0)
    masked = jnp.where(info == 1.0, iota, M)
    first_one = jnp.min(masked, axis=0, keepdims=True)
    self_idx = jnp.where(first_one == M, 0, first_one)

    at_self = iota == self_idx
    info_i = info.astype(jnp.int32)
    hiidx_ref[...] = jnp.sum(jnp.where(at_self, info_i, 0), axis=0, keepdims=True)
    xi_x = jnp.sum(jnp.where(at_self, rx, 0.0), axis=0, keepdims=True)
    xi_y = jnp.sum(jnp.where(at_self, ry, 0.0), axis=0, keepdims=True)
    xi_ref[...] = jnp.concatenate([xi_x, xi_y], axis=0)

    keep_left = lax.broadcasted_iota(jnp.int32, (M1, TB), 0) < self_idx
    src = jnp.where(keep_left, info_i[:M1], info_i[1:])
    src_ref[...] = jnp.concatenate([src, src[:1]], axis=0)
    xjx_ref[...] = jnp.where(keep_left, rx[:M1], rx[1:])
    xjy_ref[...] = jnp.where(keep_left, ry[:M1], ry[1:])


def _tc_prepare(obs3t):
    grid = (L_ROWS // TB,)
    return pl.pallas_call(
        _tc_body,
        grid=grid,
        in_specs=[pl.BlockSpec((5, M, TB), lambda i: (0, 0, i))],
        out_specs=[
            pl.BlockSpec((M, TB), lambda i: (0, i)),
            pl.BlockSpec((1, TB), lambda i: (0, i)),
            pl.BlockSpec((2, TB), lambda i: (0, i)),
            pl.BlockSpec((M1, TB), lambda i: (0, i)),
            pl.BlockSpec((M1, TB), lambda i: (0, i)),
        ],
        out_shape=[
            jax.ShapeDtypeStruct((M, L_ROWS), jnp.int32),
            jax.ShapeDtypeStruct((1, L_ROWS), jnp.int32),
            jax.ShapeDtypeStruct((2, L_ROWS), jnp.float32),
            jax.ShapeDtypeStruct((M1, L_ROWS), jnp.float32),
            jax.ShapeDtypeStruct((M1, L_ROWS), jnp.float32),
        ],
    )(obs3t)


def _transpose_chunk(gbuf, tbuf, eidx, zeros):
    def blk(i8, _):
        for bs in range(8):
            b = i8 * 8 + bs
            bvec = zeros + b
            for h in range(2):
                v = gbuf[b, pl.ds(h * 16, 16)]
                e = eidx + h * 16
                plsc.store_scatter(tbuf, [e >> 3, e & 7, bvec], v)
        return 0

    lax.fori_loop(0, CHUNK // 8, blk, 0)


def _sc_body(table_hbm, srct_hbm, hiidx_hbm, hj_hbm, hi_hbm,
             idx_v, cidx_v, gbuf, tbuf, hgbuf, gsem, ssem):
    wid = lax.axis_index("s") * NC + lax.axis_index("c")
    eidx = lax.iota(jnp.int32, 16)
    zeros = jnp.zeros((16,), jnp.int32)

    pltpu.sync_copy(srct_hbm.at[pl.ds(wid * PPW, PPW)], idx_v)

    qbase = wid * PPW

    def gdesc(t, gslot):
        return pltpu.make_async_copy(
            table_hbm.at[idx_v.at[t]], gbuf.at[gslot], gsem)

    def sdesc(t, tslot):
        q = qbase + t
        return pltpu.make_async_copy(
            tbuf.at[tslot, :, :, pl.ds(0, 128)],
            hj_hbm.at[q >> 5, :, q & 31], ssem)

    for u in range(4):
        gdesc(u, u).start()

    def step(k, _):
        for u in range(4):
            t = 4 * k + u
            live = t <= PPW - 1

            @pl.when(live)
            def _():
                gdesc(t, u).wait()

            @pl.when(jnp.logical_and(t >= 2, live))
            def _():
                sdesc(t - 2, u % 2).wait()

            @pl.when(live)
            def _():
                _transpose_chunk(gbuf.at[u], tbuf.at[u % 2], eidx, zeros)
                sdesc(t, u % 2).start()

            @pl.when(t + 4 <= PPW - 1)
            def _():
                gdesc(t + 4, u).start()

        return 0

    lax.fori_loop(0, (PPW + 3) // 4, step, 0)

    for t in (PPW - 2, PPW - 1):
        sdesc(t, t % 2).wait()

    pltpu.sync_copy(hiidx_hbm.at[wid], cidx_v)
    pltpu.async_copy(table_hbm.at[cidx_v], hgbuf, gsem).wait()
    _transpose_chunk(hgbuf, tbuf.at[0], eidx, zeros)
    pltpu.make_async_copy(tbuf.at[0, :, :, pl.ds(0, 128)],
                          hi_hbm.at[:, wid], ssem).start()
    pltpu.make_async_copy(tbuf.at[0, :, :, pl.ds(0, 128)],
                          hi_hbm.at[:, wid], ssem).wait()


@functools.cache
def _sc_gather():
    return pl.kernel(
        _sc_body,
        out_type=[
            jax.ShapeDtypeStruct((M1, NTILE, NCB, 8, 128), jnp.float32),
            jax.ShapeDtypeStruct((NTILE, NCB, 8, 128), jnp.float32),
        ],
        mesh=plsc.VectorSubcoreMesh(core_axis_name="c", subcore_axis_name="s",
                                    num_cores=NC, num_subcores=NS),
        compiler_params=pltpu.CompilerParams(use_tc_tiling_on_sc=False,
                                             needs_layout_passes=False),
        scratch_types=[
            pltpu.VMEM((PPW, CHUNK), jnp.int32),
            pltpu.VMEM((CHUNK,), jnp.int32),
            pltpu.VMEM((4, CHUNK, EMB_DIM), jnp.float32),
            pltpu.VMEM((2, NTILE, 8, 129), jnp.float32),
            pltpu.VMEM((CHUNK, EMB_DIM), jnp.float32),
            pltpu.SemaphoreType.DMA,
            pltpu.SemaphoreType.DMA,
        ],
    )


def kernel(obs, emb_weight):
    obs3t = jnp.transpose(obs.reshape(L_ROWS, 5, M), (1, 2, 0))
    srct, hiidx, xi_t, xjxt, xjyt = _tc_prepare(obs3t)
    srct3 = srct.reshape(M * NCB, CHUNK)
    hiidx2 = hiidx.reshape(NW, CHUNK)
    hj5, hi4 = _sc_gather()(emb_weight, srct3, hiidx2)
    hj = hj5.transpose(2, 4, 0, 1, 3).reshape(L_ROWS, M1, EMB_DIM)
    hi = hi4.transpose(1, 3, 0, 2).reshape(L_ROWS, EMB_DIM)
    xi = xi_t.T
    xj = jnp.stack([xjxt, xjyt], axis=1).transpose(2, 0, 1)
    return (hi, xi, hj, xj)
